# baseline (speedup 1.0000x reference)
import functools

import jax
import jax.numpy as jnp
from jax import lax
from jax.experimental import pallas as pl
from jax.experimental.pallas import tpu as pltpu
from jax.experimental.pallas import tpu_sc as plsc

N = 10000
D_IN = 128
D_H = 16
E = 320000

NC = 2
NS = 16
NW = NC * NS
C = 128
NCHUNK = 78
CT = 16
E_MAIN = NW * NCHUNK * C
PN = 10240
RPT = PN // NS
SPT = PN // NW
LAG = 4

_sc_mesh = plsc.VectorSubcoreMesh(core_axis_name="c", subcore_axis_name="s")
_sc_params = pltpu.CompilerParams(use_tc_tiling_on_sc=False)
_sc_params_nl = pltpu.CompilerParams(use_tc_tiling_on_sc=False,
                                     needs_layout_passes=False)


@functools.partial(
    pl.kernel,
    out_type=jax.ShapeDtypeStruct((NC, 2, PN), jnp.float32),
    mesh=_sc_mesh,
    scratch_types=[
        pltpu.VMEM((NCHUNK, C), jnp.int32),
        pltpu.VMEM((NCHUNK, C), jnp.int32),
        pltpu.VMEM((CT,), jnp.int32),
        pltpu.VMEM((CT,), jnp.int32),
        pltpu.VMEM((C,), jnp.float32),
        pltpu.VMEM((RPT,), jnp.float32),
        pltpu.VMEM_SHARED((PN,), jnp.float32),
        pltpu.VMEM_SHARED((PN,), jnp.float32),
        pltpu.SemaphoreType.DMA,
        pltpu.SemaphoreType.DMA,
    ],
    compiler_params=_sc_params,
)
def _deg_kernel(srcs, dsts, srcs_t, dsts_t, ones_hbm, zeros_hbm, degp,
                idxs_v, idxd_v, idxst_v, idxdt_v, ones_v, buf_v,
                degs_sh, degd_sh, sem0, sem1):
    c = lax.axis_index("c")
    s = lax.axis_index("s")
    wid = s * NC + c
    base = s * RPT
    pltpu.sync_copy(ones_hbm, ones_v)
    pltpu.sync_copy(zeros_hbm.at[pl.ds(base, RPT)], buf_v)
    pltpu.sync_copy(buf_v, degs_sh.at[pl.ds(base, RPT)])
    pltpu.sync_copy(buf_v, degd_sh.at[pl.ds(base, RPT)])
    plsc.subcore_barrier()
    pltpu.sync_copy(srcs.at[wid], idxs_v)
    pltpu.sync_copy(dsts.at[wid], idxd_v)
    pltpu.sync_copy(srcs_t.at[wid], idxst_v)
    pltpu.sync_copy(dsts_t.at[wid], idxdt_v)

    def body(j, carry):
        pltpu.async_copy(ones_v, degs_sh.at[idxs_v.at[j]], sem0, add=True)
        pltpu.async_copy(ones_v, degd_sh.at[idxd_v.at[j]], sem1, add=True)

        @pl.when(j >= LAG)
        def _():
            k = j - LAG
            pltpu.make_async_copy(ones_v, degs_sh.at[idxs_v.at[k]], sem0).wait()
            pltpu.make_async_copy(ones_v, degd_sh.at[idxd_v.at[k]], sem1).wait()

        return carry

    lax.fori_loop(0, NCHUNK, body, 0)

    def drain(j, carry):
        pltpu.make_async_copy(ones_v, degs_sh.at[idxs_v.at[j]], sem0).wait()
        pltpu.make_async_copy(ones_v, degd_sh.at[idxd_v.at[j]], sem1).wait()
        return carry

    lax.fori_loop(NCHUNK - LAG, NCHUNK, drain, 0)
    pltpu.sync_copy(ones_v.at[pl.ds(0, CT)], degs_sh.at[idxst_v], add=True)
    pltpu.sync_copy(ones_v.at[pl.ds(0, CT)], degd_sh.at[idxdt_v], add=True)
    plsc.subcore_barrier()
    pltpu.sync_copy(degs_sh.at[pl.ds(base, RPT)], buf_v)
    pltpu.sync_copy(buf_v, degp.at[c, 0, pl.ds(base, RPT)])
    pltpu.sync_copy(degd_sh.at[pl.ds(base, RPT)], buf_v)
    pltpu.sync_copy(buf_v, degp.at[c, 1, pl.ds(base, RPT)])


@functools.partial(
    pl.kernel,
    out_type=jax.ShapeDtypeStruct((NC, PN, D_H), jnp.float32),
    mesh=_sc_mesh,
    scratch_types=[
        pltpu.VMEM((NCHUNK, C), jnp.int32),
        pltpu.VMEM((NCHUNK, C), jnp.int32),
        pltpu.VMEM((CT,), jnp.int32),
        pltpu.VMEM((CT,), jnp.int32),
        *[pltpu.VMEM((C, D_H), jnp.float32) for _ in range(8)],
        pltpu.VMEM((CT, D_H), jnp.float32),
        pltpu.VMEM((RPT, D_H), jnp.float32),
        pltpu.VMEM_SHARED((PN, D_H), jnp.float32),
        *[pltpu.SemaphoreType.DMA for _ in range(16)],
    ],
    compiler_params=_sc_params,
)
def _agg_kernel(x, srcs, dsts, srcs_t, dsts_t, zeros_hbm, aggp,
                idxs_v, idxd_v, idxst_v, idxdt_v,
                r0, r1, r2, r3, r4, r5, r6, r7, rowst,
                buf_v, acc_sh,
                g0, g1, g2, g3, g4, g5, g6, g7,
                s0, s1, s2, s3, s4, s5, s6, s7):
    c = lax.axis_index("c")
    s = lax.axis_index("s")
    wid = s * NC + c
    base = s * RPT
    rows = (r0, r1, r2, r3, r4, r5, r6, r7)
    gs = (g0, g1, g2, g3, g4, g5, g6, g7)
    ss = (s0, s1, s2, s3, s4, s5, s6, s7)
    NB, PF = 8, 4
    pltpu.sync_copy(zeros_hbm.at[pl.ds(base, RPT)], buf_v)
    pltpu.sync_copy(buf_v, acc_sh.at[pl.ds(base, RPT)])
    plsc.subcore_barrier()
    pltpu.sync_copy(srcs.at[wid], idxs_v)
    pltpu.sync_copy(dsts.at[wid], idxd_v)
    pltpu.sync_copy(srcs_t.at[wid], idxst_v)
    pltpu.sync_copy(dsts_t.at[wid], idxdt_v)
    for p in range(PF):
        pltpu.async_copy(x.at[idxs_v.at[p]], rows[p], gs[p])

    def body(j, carry):
        for p in range(NB):
            @pl.when(j % NB == p)
            def _(p=p):
                q = (p + PF) % NB
                pltpu.make_async_copy(x.at[idxs_v.at[j]], rows[p], gs[p]).wait()
                pltpu.async_copy(rows[p], acc_sh.at[idxd_v.at[j]], ss[p],
                                 add=True)

                @pl.when(j + PF < NCHUNK)
                def _():
                    @pl.when(j >= PF)
                    def _():
                        pltpu.make_async_copy(
                            rows[q], acc_sh.at[idxd_v.at[j - PF]], ss[q]).wait()

                    pltpu.async_copy(x.at[idxs_v.at[j + PF]], rows[q], gs[q])

        return carry

    lax.fori_loop(0, NCHUNK, body, 0)
    for k in range(NCHUNK - 2 * PF, NCHUNK):
        pltpu.make_async_copy(rows[k % NB], acc_sh.at[idxd_v.at[k]],
                              ss[k % NB]).wait()
    pltpu.async_copy(x.at[idxst_v], rowst, gs[0]).wait()
    pltpu.sync_copy(rowst, acc_sh.at[idxdt_v], add=True)
    plsc.subcore_barrier()
    pltpu.sync_copy(acc_sh.at[pl.ds(base, RPT)], buf_v)
    pltpu.sync_copy(buf_v, aggp.at[c, pl.ds(base, RPT)])


@functools.partial(
    pl.kernel,
    out_type=[
        jax.ShapeDtypeStruct((PN, D_H), jnp.float32),
        jax.ShapeDtypeStruct((PN, D_H), jnp.float32),
    ],
    mesh=_sc_mesh,
    scratch_types=[
        pltpu.VMEM((SPT, D_H), jnp.float32),
        pltpu.VMEM((SPT, D_H), jnp.float32),
        pltpu.VMEM((SPT,), jnp.float32),
        pltpu.VMEM((SPT,), jnp.float32),
        pltpu.VMEM((SPT,), jnp.float32),
        pltpu.VMEM((D_H,), jnp.float32),
        pltpu.VMEM((SPT, D_H), jnp.float32),
        pltpu.VMEM((SPT, D_H), jnp.float32),
    ],
    compiler_params=_sc_params_nl,
)
def _combine_kernel(aggp, m, so, si, b1, feat2, sip,
                    p0_v, p1_v, m_v, so_v, si_v, b1_v, o_v, sp_v):
    c = lax.axis_index("c")
    s = lax.axis_index("s")
    wid = s * NC + c
    base = wid * SPT
    pltpu.sync_copy(aggp.at[0, pl.ds(base, SPT)], p0_v)
    pltpu.sync_copy(aggp.at[1, pl.ds(base, SPT)], p1_v)
    pltpu.sync_copy(m.at[pl.ds(base, SPT)], m_v)
    pltpu.sync_copy(so.at[pl.ds(base, SPT)], so_v)
    pltpu.sync_copy(si.at[pl.ds(base, SPT)], si_v)
    pltpu.sync_copy(b1, b1_v)
    bvec = b1_v[...]

    def body(t, carry):
        r = t * 2
        ridx0 = jnp.full((D_H,), r, jnp.int32)
        ridx1 = jnp.full((D_H,), r + 1, jnp.int32)
        mrow0 = plsc.load_gather(m_v, [ridx0])
        srow0 = plsc.load_gather(so_v, [ridx0])
        mrow1 = plsc.load_gather(m_v, [ridx1])
        srow1 = plsc.load_gather(so_v, [ridx1])
        o_v[r] = (p0_v[r] + p1_v[r]) * mrow0 + bvec * srow0
        o_v[r + 1] = (p0_v[r + 1] + p1_v[r + 1]) * mrow1 + bvec * srow1
        sp_v[r] = plsc.load_gather(si_v, [ridx0])
        sp_v[r + 1] = plsc.load_gather(si_v, [ridx1])
        return carry

    lax.fori_loop(0, SPT // 2, body, 0)
    pltpu.sync_copy(o_v, feat2.at[pl.ds(base, SPT)])
    pltpu.sync_copy(sp_v, sip.at[pl.ds(base, SPT)])


RB = PN


def _k1a_body(feat, w1, y):
    y[...] = jnp.dot(feat[...], w1[...], preferred_element_type=jnp.float32)


def _k1b_body(degp, y, x1, m, so, si):
    dp = degp[...]
    sc = lax.rsqrt(jnp.maximum(dp[0] + dp[1], 1.0))
    so[...] = sc[0]
    si[...] = sc[1]
    m[...] = sc[0] * sc[1]
    sct = jnp.transpose(sc)
    x1[...] = y[...] * sct[:, 0:1]


def _k5_body(q0p, q1p, sip, w2b, b2b, wlb, blb, hp, op):
    g = jnp.dot(q0p[...] + q1p[...], w2b[...],
                preferred_element_type=jnp.float32) * sip[...] + b2b[...]
    hp[...] = g
    op[...] = jnp.dot(g, wlb[...], preferred_element_type=jnp.float32) + blb[...]


def kernel(feature, edge_index, W1, b1, W2, b2, Wl, bl):
    srcs = edge_index[0, :E_MAIN].reshape(NW, NCHUNK, C)
    dsts = edge_index[1, :E_MAIN].reshape(NW, NCHUNK, C)
    srcs_t = edge_index[0, E_MAIN:].reshape(NW, CT)
    dsts_t = edge_index[1, E_MAIN:].reshape(NW, CT)
    ones = jnp.ones((C,), jnp.float32)
    zeros1 = jnp.zeros((PN,), jnp.float32)
    zeros2 = jnp.zeros((PN, D_H), jnp.float32)

    y = pl.pallas_call(
        _k1a_body,
        grid=(PN // RB,),
        in_specs=[
            pl.BlockSpec((RB, D_IN), lambda i: (i, 0)),
            pl.BlockSpec((D_IN, D_H), lambda i: (0, 0)),
        ],
        out_specs=pl.BlockSpec((RB, D_H), lambda i: (i, 0)),
        out_shape=jax.ShapeDtypeStruct((PN, D_H), jnp.float32),
    )(feature, W1)

    degp = _deg_kernel(srcs, dsts, srcs_t, dsts_t, ones, zeros1)

    x1, m, so, si = pl.pallas_call(
        _k1b_body,
        grid=(PN // RB,),
        in_specs=[
            pl.BlockSpec((NC, 2, RB), lambda i: (0, 0, i)),
            pl.BlockSpec((RB, D_H), lambda i: (i, 0)),
        ],
        out_specs=[
            pl.BlockSpec((RB, D_H), lambda i: (i, 0)),
            pl.BlockSpec((RB,), lambda i: (i,)),
            pl.BlockSpec((RB,), lambda i: (i,)),
            pl.BlockSpec((RB,), lambda i: (i,)),
        ],
        out_shape=[
            jax.ShapeDtypeStruct((PN, D_H), jnp.float32),
            jax.ShapeDtypeStruct((PN,), jnp.float32),
            jax.ShapeDtypeStruct((PN,), jnp.float32),
            jax.ShapeDtypeStruct((PN,), jnp.float32),
        ],
    )(degp, y)

    aggp1 = _agg_kernel(x1, srcs, dsts, srcs_t, dsts_t, zeros2)
    feat2, sip = _combine_kernel(aggp1, m, so, si, b1)
    aggp2 = _agg_kernel(feat2, srcs, dsts, srcs_t, dsts_t, zeros2)

    PR = PN // 8
    eye8 = jnp.eye(8, dtype=jnp.float32)
    w2b = jnp.kron(eye8, W2)
    wlb = jnp.kron(eye8, Wl)
    b2b = jnp.tile(b2, 8).reshape(1, 8 * D_H)
    blb = jnp.tile(bl, 8).reshape(1, 16)
    q0p = aggp2[0].reshape(PR, 8 * D_H)
    q1p = aggp2[1].reshape(PR, 8 * D_H)
    sipp = sip.reshape(PR, 8 * D_H)

    hp, op = pl.pallas_call(
        _k5_body,
        grid=(1,),
        in_specs=[
            pl.BlockSpec((PR, 8 * D_H), lambda i: (0, 0)),
            pl.BlockSpec((PR, 8 * D_H), lambda i: (0, 0)),
            pl.BlockSpec((PR, 8 * D_H), lambda i: (0, 0)),
            pl.BlockSpec((8 * D_H, 8 * D_H), lambda i: (0, 0)),
            pl.BlockSpec((1, 8 * D_H), lambda i: (0, 0)),
            pl.BlockSpec((8 * D_H, 16), lambda i: (0, 0)),
            pl.BlockSpec((1, 16), lambda i: (0, 0)),
        ],
        out_specs=[
            pl.BlockSpec((PR, 8 * D_H), lambda i: (0, 0)),
            pl.BlockSpec((PR, 16), lambda i: (0, 0)),
        ],
        out_shape=[
            jax.ShapeDtypeStruct((PR, 8 * D_H), jnp.float32),
            jax.ShapeDtypeStruct((PR, 16), jnp.float32),
        ],
    )(q0p, q1p, sipp, w2b, b2b, wlb, blb)

    h = hp.reshape(PN, D_H)[:N]
    out = op.reshape(PN, 2)[:N]
    return (h, out)


# speedup vs baseline: 1.1493x; 1.1493x over previous
import functools

import jax
import jax.numpy as jnp
from jax import lax
from jax.experimental import pallas as pl
from jax.experimental.pallas import tpu as pltpu
from jax.experimental.pallas import tpu_sc as plsc

N = 10000
D_IN = 128
D_H = 16
E = 320000

NC = 2
NS = 16
NW = NC * NS
C = 128
NCHUNK = 78
CT = 16
E_MAIN = NW * NCHUNK * C
PN = 10240
RPT = PN // NS
SPT = PN // NW
LAG = 4

_sc_mesh = plsc.VectorSubcoreMesh(core_axis_name="c", subcore_axis_name="s")
_sc_params = pltpu.CompilerParams(use_tc_tiling_on_sc=False)
_sc_params_nl = pltpu.CompilerParams(use_tc_tiling_on_sc=False,
                                     needs_layout_passes=False)


@functools.partial(
    pl.kernel,
    out_type=jax.ShapeDtypeStruct((NC, 2, PN), jnp.float32),
    mesh=_sc_mesh,
    scratch_types=[
        pltpu.VMEM((NCHUNK, C), jnp.int32),
        pltpu.VMEM((NCHUNK, C), jnp.int32),
        pltpu.VMEM((CT,), jnp.int32),
        pltpu.VMEM((CT,), jnp.int32),
        pltpu.VMEM((C,), jnp.float32),
        pltpu.VMEM((RPT,), jnp.float32),
        pltpu.VMEM_SHARED((PN,), jnp.float32),
        pltpu.VMEM_SHARED((PN,), jnp.float32),
        pltpu.SemaphoreType.DMA,
        pltpu.SemaphoreType.DMA,
    ],
    compiler_params=_sc_params,
)
def _deg_kernel(srcs, dsts, srcs_t, dsts_t, ones_hbm, zeros_hbm, degp,
                idxs_v, idxd_v, idxst_v, idxdt_v, ones_v, buf_v,
                degs_sh, degd_sh, sem0, sem1):
    c = lax.axis_index("c")
    s = lax.axis_index("s")
    wid = s * NC + c
    base = s * RPT
    pltpu.sync_copy(ones_hbm, ones_v)
    pltpu.sync_copy(zeros_hbm.at[pl.ds(base, RPT)], buf_v)
    pltpu.sync_copy(buf_v, degs_sh.at[pl.ds(base, RPT)])
    pltpu.sync_copy(buf_v, degd_sh.at[pl.ds(base, RPT)])
    plsc.subcore_barrier()
    pltpu.sync_copy(srcs.at[wid], idxs_v)
    pltpu.sync_copy(dsts.at[wid], idxd_v)
    pltpu.sync_copy(srcs_t.at[wid], idxst_v)
    pltpu.sync_copy(dsts_t.at[wid], idxdt_v)

    def body(j, carry):
        pltpu.async_copy(ones_v, degs_sh.at[idxs_v.at[j]], sem0, add=True)
        pltpu.async_copy(ones_v, degd_sh.at[idxd_v.at[j]], sem1, add=True)

        @pl.when(j >= LAG)
        def _():
            k = j - LAG
            pltpu.make_async_copy(ones_v, degs_sh.at[idxs_v.at[k]], sem0).wait()
            pltpu.make_async_copy(ones_v, degd_sh.at[idxd_v.at[k]], sem1).wait()

        return carry

    lax.fori_loop(0, NCHUNK, body, 0)

    def drain(j, carry):
        pltpu.make_async_copy(ones_v, degs_sh.at[idxs_v.at[j]], sem0).wait()
        pltpu.make_async_copy(ones_v, degd_sh.at[idxd_v.at[j]], sem1).wait()
        return carry

    lax.fori_loop(NCHUNK - LAG, NCHUNK, drain, 0)
    pltpu.sync_copy(ones_v.at[pl.ds(0, CT)], degs_sh.at[idxst_v], add=True)
    pltpu.sync_copy(ones_v.at[pl.ds(0, CT)], degd_sh.at[idxdt_v], add=True)
    plsc.subcore_barrier()
    pltpu.sync_copy(degs_sh.at[pl.ds(base, RPT)], buf_v)
    pltpu.sync_copy(buf_v, degp.at[c, 0, pl.ds(base, RPT)])
    pltpu.sync_copy(degd_sh.at[pl.ds(base, RPT)], buf_v)
    pltpu.sync_copy(buf_v, degp.at[c, 1, pl.ds(base, RPT)])


@functools.partial(
    pl.kernel,
    out_type=jax.ShapeDtypeStruct((NC, PN, D_H), jnp.float32),
    mesh=_sc_mesh,
    scratch_types=[
        pltpu.VMEM((NCHUNK, C), jnp.int32),
        pltpu.VMEM((NCHUNK, C), jnp.int32),
        pltpu.VMEM((CT,), jnp.int32),
        pltpu.VMEM((CT,), jnp.int32),
        *[pltpu.VMEM((C, D_H), jnp.float32) for _ in range(8)],
        pltpu.VMEM((CT, D_H), jnp.float32),
        pltpu.VMEM((RPT, D_H), jnp.float32),
        pltpu.VMEM_SHARED((PN, D_H), jnp.float32),
        *[pltpu.SemaphoreType.DMA for _ in range(16)],
    ],
    compiler_params=_sc_params,
)
def _agg_kernel(x, srcs, dsts, srcs_t, dsts_t, zeros_hbm, aggp,
                idxs_v, idxd_v, idxst_v, idxdt_v,
                r0, r1, r2, r3, r4, r5, r6, r7, rowst,
                buf_v, acc_sh,
                g0, g1, g2, g3, g4, g5, g6, g7,
                s0, s1, s2, s3, s4, s5, s6, s7):
    c = lax.axis_index("c")
    s = lax.axis_index("s")
    wid = s * NC + c
    base = s * RPT
    rows = (r0, r1, r2, r3, r4, r5, r6, r7)
    gs = (g0, g1, g2, g3, g4, g5, g6, g7)
    ss = (s0, s1, s2, s3, s4, s5, s6, s7)
    NB, PF = 8, 4
    pltpu.sync_copy(zeros_hbm.at[pl.ds(base, RPT)], buf_v)
    pltpu.sync_copy(buf_v, acc_sh.at[pl.ds(base, RPT)])
    plsc.subcore_barrier()
    pltpu.sync_copy(srcs.at[wid], idxs_v)
    pltpu.sync_copy(dsts.at[wid], idxd_v)
    pltpu.sync_copy(srcs_t.at[wid], idxst_v)
    pltpu.sync_copy(dsts_t.at[wid], idxdt_v)
    for p in range(PF):
        pltpu.async_copy(x.at[idxs_v.at[p]], rows[p], gs[p])

    def body(j, carry):
        for p in range(NB):
            @pl.when(j % NB == p)
            def _(p=p):
                q = (p + PF) % NB
                pltpu.make_async_copy(x.at[idxs_v.at[j]], rows[p], gs[p]).wait()
                pltpu.async_copy(rows[p], acc_sh.at[idxd_v.at[j]], ss[p],
                                 add=True)

                @pl.when(j + PF < NCHUNK)
                def _():
                    @pl.when(j >= PF)
                    def _():
                        pltpu.make_async_copy(
                            rows[q], acc_sh.at[idxd_v.at[j - PF]], ss[q]).wait()

                    pltpu.async_copy(x.at[idxs_v.at[j + PF]], rows[q], gs[q])

        return carry

    lax.fori_loop(0, NCHUNK, body, 0)
    for k in range(NCHUNK - 2 * PF, NCHUNK):
        pltpu.make_async_copy(rows[k % NB], acc_sh.at[idxd_v.at[k]],
                              ss[k % NB]).wait()
    pltpu.async_copy(x.at[idxst_v], rowst, gs[0]).wait()
    pltpu.sync_copy(rowst, acc_sh.at[idxdt_v], add=True)
    plsc.subcore_barrier()
    pltpu.sync_copy(acc_sh.at[pl.ds(base, RPT)], buf_v)
    pltpu.sync_copy(buf_v, aggp.at[c, pl.ds(base, RPT)])


def _rsqrt16(d):
    i = plsc.bitcast(d, jnp.int32)
    i = jnp.int32(0x5F3759DF) - lax.shift_right_arithmetic(i, 1)
    z = plsc.bitcast(i, jnp.float32)
    h = d * 0.5
    for _ in range(3):
        z = z * (1.5 - h * z * z)
    return z


@functools.partial(
    pl.kernel,
    out_type=[
        jax.ShapeDtypeStruct((PN, D_H), jnp.float32),
        jax.ShapeDtypeStruct((PN,), jnp.float32),
        jax.ShapeDtypeStruct((PN,), jnp.float32),
        jax.ShapeDtypeStruct((PN,), jnp.float32),
    ],
    mesh=_sc_mesh,
    scratch_types=[
        pltpu.VMEM((SPT, D_H), jnp.float32),
        pltpu.VMEM((SPT,), jnp.float32),
        pltpu.VMEM((SPT,), jnp.float32),
        pltpu.VMEM((SPT,), jnp.float32),
        pltpu.VMEM((SPT,), jnp.float32),
        pltpu.VMEM((SPT,), jnp.float32),
        pltpu.VMEM((SPT,), jnp.float32),
        pltpu.VMEM((SPT,), jnp.float32),
        pltpu.VMEM((SPT, D_H), jnp.float32),
    ],
    compiler_params=_sc_params_nl,
)
def _scale_kernel(y, degp, x1, m, so, si,
                  y_v, d00, d01, d10, d11, m_v, so_v, si_v, x1_v):
    c = lax.axis_index("c")
    s = lax.axis_index("s")
    wid = s * NC + c
    base = wid * SPT
    pltpu.sync_copy(y.at[pl.ds(base, SPT)], y_v)
    pltpu.sync_copy(degp.at[0, 0, pl.ds(base, SPT)], d00)
    pltpu.sync_copy(degp.at[0, 1, pl.ds(base, SPT)], d01)
    pltpu.sync_copy(degp.at[1, 0, pl.ds(base, SPT)], d10)
    pltpu.sync_copy(degp.at[1, 1, pl.ds(base, SPT)], d11)

    def scales(t, carry):
        sl = pl.ds(t * 16, 16)
        dout = jnp.maximum(d00[sl] + d10[sl], 1.0)
        din = jnp.maximum(d01[sl] + d11[sl], 1.0)
        so_c = _rsqrt16(dout)
        si_c = _rsqrt16(din)
        so_v[sl] = so_c
        si_v[sl] = si_c
        m_v[sl] = so_c * si_c
        return carry

    lax.fori_loop(0, SPT // 16, scales, 0)

    def rows(t, carry):
        r = t * 2
        srow0 = plsc.load_gather(so_v, [jnp.full((D_H,), r, jnp.int32)])
        srow1 = plsc.load_gather(so_v, [jnp.full((D_H,), r + 1, jnp.int32)])
        x1_v[r] = y_v[r] * srow0
        x1_v[r + 1] = y_v[r + 1] * srow1
        return carry

    lax.fori_loop(0, SPT // 2, rows, 0)
    pltpu.sync_copy(x1_v, x1.at[pl.ds(base, SPT)])
    pltpu.sync_copy(m_v, m.at[pl.ds(base, SPT)])
    pltpu.sync_copy(so_v, so.at[pl.ds(base, SPT)])
    pltpu.sync_copy(si_v, si.at[pl.ds(base, SPT)])


@functools.partial(
    pl.kernel,
    out_type=jax.ShapeDtypeStruct((PN, D_H), jnp.float32),
    mesh=_sc_mesh,
    scratch_types=[
        pltpu.VMEM((SPT, D_H), jnp.float32),
        pltpu.VMEM((SPT, D_H), jnp.float32),
        pltpu.VMEM((SPT,), jnp.float32),
        pltpu.VMEM((SPT,), jnp.float32),
        pltpu.VMEM((D_H,), jnp.float32),
        pltpu.VMEM((SPT, D_H), jnp.float32),
    ],
    compiler_params=_sc_params_nl,
)
def _combine_kernel(aggp, m, so, b1, feat2, p0_v, p1_v, m_v, so_v, b1_v, o_v):
    c = lax.axis_index("c")
    s = lax.axis_index("s")
    wid = s * NC + c
    base = wid * SPT
    pltpu.sync_copy(aggp.at[0, pl.ds(base, SPT)], p0_v)
    pltpu.sync_copy(aggp.at[1, pl.ds(base, SPT)], p1_v)
    pltpu.sync_copy(m.at[pl.ds(base, SPT)], m_v)
    pltpu.sync_copy(so.at[pl.ds(base, SPT)], so_v)
    pltpu.sync_copy(b1, b1_v)
    bvec = b1_v[...]

    def body(t, carry):
        r = t * 2
        ridx0 = jnp.full((D_H,), r, jnp.int32)
        ridx1 = jnp.full((D_H,), r + 1, jnp.int32)
        mrow0 = plsc.load_gather(m_v, [ridx0])
        srow0 = plsc.load_gather(so_v, [ridx0])
        mrow1 = plsc.load_gather(m_v, [ridx1])
        srow1 = plsc.load_gather(so_v, [ridx1])
        o_v[r] = (p0_v[r] + p1_v[r]) * mrow0 + bvec * srow0
        o_v[r + 1] = (p0_v[r + 1] + p1_v[r + 1]) * mrow1 + bvec * srow1
        return carry

    lax.fori_loop(0, SPT // 2, body, 0)
    pltpu.sync_copy(o_v, feat2.at[pl.ds(base, SPT)])


RB = PN


def _k1a_body(feat, w1, y):
    y[...] = jnp.dot(feat[...], w1[...], preferred_element_type=jnp.float32)


def _k5_body(q, si, w2, b2, wl, bl, h, out):
    qs = q[...]
    si_col = si[...][:, None]
    g = jnp.dot(qs[0] + qs[1], w2[...],
                preferred_element_type=jnp.float32) * si_col + b2[...]
    h[...] = g
    out[...] = jnp.dot(g, wl[...], preferred_element_type=jnp.float32) + bl[...]


def kernel(feature, edge_index, W1, b1, W2, b2, Wl, bl):
    srcs = edge_index[0, :E_MAIN].reshape(NW, NCHUNK, C)
    dsts = edge_index[1, :E_MAIN].reshape(NW, NCHUNK, C)
    srcs_t = edge_index[0, E_MAIN:].reshape(NW, CT)
    dsts_t = edge_index[1, E_MAIN:].reshape(NW, CT)
    ones = jnp.ones((C,), jnp.float32)
    zeros1 = jnp.zeros((PN,), jnp.float32)
    zeros2 = jnp.zeros((PN, D_H), jnp.float32)

    y = pl.pallas_call(
        _k1a_body,
        grid=(PN // RB,),
        in_specs=[
            pl.BlockSpec((RB, D_IN), lambda i: (i, 0)),
            pl.BlockSpec((D_IN, D_H), lambda i: (0, 0)),
        ],
        out_specs=pl.BlockSpec((RB, D_H), lambda i: (i, 0)),
        out_shape=jax.ShapeDtypeStruct((PN, D_H), jnp.float32),
    )(feature, W1)

    degp = _deg_kernel(srcs, dsts, srcs_t, dsts_t, ones, zeros1)
    x1, m, so, si = _scale_kernel(y, degp)

    aggp1 = _agg_kernel(x1, srcs, dsts, srcs_t, dsts_t, zeros2)
    feat2 = _combine_kernel(aggp1, m, so, b1)
    aggp2 = _agg_kernel(feat2, srcs, dsts, srcs_t, dsts_t, zeros2)

    h, out = pl.pallas_call(
        _k5_body,
        grid=(PN // RB,),
        in_specs=[
            pl.BlockSpec((NC, RB, D_H), lambda i: (0, i, 0)),
            pl.BlockSpec((RB,), lambda i: (i,)),
            pl.BlockSpec((D_H, D_H), lambda i: (0, 0)),
            pl.BlockSpec((1, D_H), lambda i: (0, 0)),
            pl.BlockSpec((D_H, 2), lambda i: (0, 0)),
            pl.BlockSpec((1, 2), lambda i: (0, 0)),
        ],
        out_specs=[
            pl.BlockSpec((RB, D_H), lambda i: (i, 0)),
            pl.BlockSpec((RB, 2), lambda i: (i, 0)),
        ],
        out_shape=[
            jax.ShapeDtypeStruct((N, D_H), jnp.float32),
            jax.ShapeDtypeStruct((N, 2), jnp.float32),
        ],
    )(aggp2, si, W2, b2.reshape(1, D_H), Wl, bl.reshape(1, 2))

    return (h, out)
